# speedup vs baseline: 3.6074x; 3.6074x over previous
import jax
import jax.numpy as jnp
from jax.experimental import pallas as pl
from jax.experimental.pallas import tpu as pltpu

N = 100000
B = 128
D = 128
BN = 1000
NB = N // BN


def _seg_kernel(h_ref, b_ref, out_ref):
    bids = b_ref[0, 0, :]
    onehot = (bids[:, None] == jax.lax.broadcasted_iota(jnp.int32, (1, B), 1)
              ).astype(jnp.float32)
    partial = jax.lax.dot_general(
        onehot, h_ref[...], (((0,), (0,)), ((), ())),
        preferred_element_type=jnp.float32)
    @pl.when(pl.program_id(0) == 0)
    def _():
        out_ref[...] = partial
    @pl.when(pl.program_id(0) != 0)
    def _():
        out_ref[...] += partial


def _bcast_kernel(h_ref, b_ref, pool_ref, vnh_ref, w_ref, bias_ref,
                  hout_ref, vnout_ref, vn_sc):
    @pl.when(pl.program_id(0) == 0)
    def _():
        x = vnh_ref[...] + pool_ref[...]
        t = jax.lax.dot_general(x, w_ref[...], (((1,), (0,)), ((), ())),
                                preferred_element_type=jnp.float32)
        vn = vnh_ref[...] + jnp.maximum(t + bias_ref[...], 0.0)
        vn_sc[...] = vn
        vnout_ref[...] = vn
    bids = b_ref[0, 0, :]
    onehot = (bids[:, None] == jax.lax.broadcasted_iota(jnp.int32, (1, B), 1)
              ).astype(jnp.float32)
    hout_ref[...] = h_ref[...] + jax.lax.dot_general(
        onehot, vn_sc[...], (((1,), (0,)), ((), ())),
        preferred_element_type=jnp.float32)


@jax.jit
def kernel(h, batch, vn_h, W, b):
    batch3 = batch.reshape(NB, 1, BN)
    bias2 = b.reshape(1, D)

    pool = pl.pallas_call(
        _seg_kernel,
        grid=(NB,),
        in_specs=[
            pl.BlockSpec((BN, D), lambda i: (i, 0)),
            pl.BlockSpec((1, 1, BN), lambda i: (i, 0, 0)),
        ],
        out_specs=pl.BlockSpec((B, D), lambda i: (0, 0)),
        out_shape=jax.ShapeDtypeStruct((B, D), jnp.float32),
    )(h, batch3)

    h_new, vn_new = pl.pallas_call(
        _bcast_kernel,
        grid=(NB,),
        in_specs=[
            pl.BlockSpec((BN, D), lambda i: (i, 0)),
            pl.BlockSpec((1, 1, BN), lambda i: (i, 0, 0)),
            pl.BlockSpec((B, D), lambda i: (0, 0)),
            pl.BlockSpec((B, D), lambda i: (0, 0)),
            pl.BlockSpec((D, D), lambda i: (0, 0)),
            pl.BlockSpec((1, D), lambda i: (0, 0)),
        ],
        out_specs=[
            pl.BlockSpec((BN, D), lambda i: (i, 0)),
            pl.BlockSpec((B, D), lambda i: (0, 0)),
        ],
        out_shape=[
            jax.ShapeDtypeStruct((N, D), jnp.float32),
            jax.ShapeDtypeStruct((B, D), jnp.float32),
        ],
        scratch_shapes=[pltpu.VMEM((B, D), jnp.float32)],
    )(h, batch3, pool, vn_h, W, bias2)

    return h_new, vn_new
